# baseline (speedup 1.0000x reference)
import functools

import jax
import jax.numpy as jnp
import numpy as np
from jax import lax
from jax.experimental import pallas as pl
from jax.experimental.pallas import tpu as pltpu
from jax.experimental.pallas import tpu_sc as plsc

C = 384
K = 5
L = 16
NSLICE = C // L
BIG = np.int32(1 << 30)
B = 16
HW = 56 * 56
CB = 96


def _sc_topk_body(bn_e_hbm, bn_n_hbm, idxs_hbm, vb_ref, tmp_ref, iout_ref):
    cid = lax.axis_index("c")
    sid = lax.axis_index("s")
    lane = lax.iota(jnp.int32, L)

    def splat_min_i32(x):
        for d in (1, 2, 4, 8):
            tmp_ref[...] = x
            x = jnp.minimum(x, plsc.load_gather(tmp_ref, [lane ^ d]))
        return x

    def splat_min_f32(x):
        for d in (1, 2, 4, 8):
            tmp_ref[...] = plsc.bitcast(x, jnp.int32)
            g = plsc.bitcast(plsc.load_gather(tmp_ref, [lane ^ d]),
                             jnp.float32)
            x = jnp.minimum(x, g)
        return x

    def top5_smallest():
        def absify(i, _):
            vb_ref[pl.ds(i * L, L)] = jnp.abs(vb_ref[pl.ds(i * L, L)])
            return 0
        lax.fori_loop(0, NSLICE, absify, 0)

        idxvec = jnp.zeros((L,), jnp.int32)
        for p in range(K):
            def scan_min(i, m):
                return jnp.minimum(m, vb_ref[pl.ds(i * L, L)])
            m = lax.fori_loop(0, NSLICE, scan_min,
                              jnp.full((L,), jnp.inf, jnp.float32))
            mmin = splat_min_f32(m)

            def scan_arg(i, best):
                v = vb_ref[pl.ds(i * L, L)]
                cand = jnp.where(v == mmin, lane + i * L, BIG)
                return jnp.minimum(best, cand)
            best = lax.fori_loop(0, NSLICE, scan_arg,
                                 jnp.full((L,), BIG, jnp.int32))
            widx = splat_min_i32(best)

            def mask_out(i, _):
                v = vb_ref[pl.ds(i * L, L)]
                vb_ref[pl.ds(i * L, L)] = jnp.where(
                    lane + i * L == widx, jnp.inf, v)
                return 0
            lax.fori_loop(0, NSLICE, mask_out, 0)
            idxvec = jnp.where(lane == p, widx, idxvec)
        return idxvec

    @pl.when(jnp.logical_and(cid == 0, sid == 0))
    def _():
        pltpu.sync_copy(bn_n_hbm, vb_ref)
        iout_ref[...] = top5_smallest()
        pltpu.sync_copy(iout_ref, idxs_hbm.at[pl.ds(L, L)])

    @pl.when(jnp.logical_and(cid != 0, sid == 0))
    def _():
        pltpu.sync_copy(bn_e_hbm, vb_ref)
        iout_ref[...] = top5_smallest()
        pltpu.sync_copy(iout_ref, idxs_hbm.at[pl.ds(0, L)])


def _sc_topk(bn_e, bn_n):
    mesh = plsc.VectorSubcoreMesh(core_axis_name="c", subcore_axis_name="s")
    fn = pl.kernel(
        _sc_topk_body,
        out_type=jax.ShapeDtypeStruct((2 * L,), jnp.int32),
        mesh=mesh,
        scratch_types=[
            pltpu.VMEM((C,), jnp.float32),
            pltpu.VMEM((L,), jnp.int32),
            pltpu.VMEM((L,), jnp.int32),
        ],
        compiler_params=pltpu.CompilerParams(needs_layout_passes=False),
    )
    return fn(bn_e, bn_n)


def _tc_exchange_body(idxs_ref, e_ref, n_ref, *rest):
    swap_refs = rest[:2 * K]
    x1_ref, x2_ref = rest[2 * K], rest[2 * K + 1]
    j = pl.program_id(1)
    base = j * CB
    x1_ref[...] = e_ref[...]
    x2_ref[...] = n_ref[...]
    for k in range(K):
        p1 = idxs_ref[k]
        p2 = idxs_ref[L + k]
        ns_ref = swap_refs[k]
        es_ref = swap_refs[K + k]

        @pl.when(jnp.logical_and(p1 >= base, p1 < base + CB))
        def _():
            x1_ref[:, pl.ds(p1 - base, 1)] = ns_ref[...].reshape(1, 1, HW)

        @pl.when(jnp.logical_and(p2 >= base, p2 < base + CB))
        def _():
            x2_ref[:, pl.ds(p2 - base, 1)] = es_ref[...].reshape(1, 1, HW)


def _tc_exchange(idxs, e3, n3):
    blk = (1, CB, HW)
    sblk = (1, 1, 1, HW)
    nswap_specs = [
        pl.BlockSpec(sblk, (lambda b, j, r, k=k: (b, r[L + k], 0, 0)))
        for k in range(K)
    ]
    eswap_specs = [
        pl.BlockSpec(sblk, (lambda b, j, r, k=k: (b, r[k], 0, 0)))
        for k in range(K)
    ]
    grid_spec = pltpu.PrefetchScalarGridSpec(
        num_scalar_prefetch=1,
        grid=(B, C // CB),
        in_specs=[
            pl.BlockSpec(blk, lambda b, j, r: (b, j, 0)),
            pl.BlockSpec(blk, lambda b, j, r: (b, j, 0)),
            *nswap_specs,
            *eswap_specs,
        ],
        out_specs=[
            pl.BlockSpec(blk, lambda b, j, r: (b, j, 0)),
            pl.BlockSpec(blk, lambda b, j, r: (b, j, 0)),
        ],
    )
    call = pl.pallas_call(
        _tc_exchange_body,
        grid_spec=grid_spec,
        out_shape=[jax.ShapeDtypeStruct(e3.shape, e3.dtype)] * 2,
        compiler_params=pltpu.CompilerParams(
            dimension_semantics=("arbitrary", "arbitrary"),
        ),
    )
    n4 = n3.reshape(B, C, 1, HW)
    e4 = e3.reshape(B, C, 1, HW)
    return call(idxs, e3, n3, *([n4] * K), *([e4] * K))


def _tc_bulk_body(e_ref, n_ref, x1_ref, x2_ref):
    x1_ref[...] = e_ref[...]
    x2_ref[...] = n_ref[...]


def _tc_bulk(e3, n3):
    blk = (1, CB, HW)
    grid_spec = pl.GridSpec(
        grid=(B, C // CB),
        in_specs=[
            pl.BlockSpec(blk, lambda b, j: (b, j, 0)),
            pl.BlockSpec(blk, lambda b, j: (b, j, 0)),
        ],
        out_specs=[
            pl.BlockSpec(blk, lambda b, j: (b, j, 0)),
            pl.BlockSpec(blk, lambda b, j: (b, j, 0)),
        ],
    )
    return pl.pallas_call(
        _tc_bulk_body,
        grid_spec=grid_spec,
        out_shape=[jax.ShapeDtypeStruct(e3.shape, e3.dtype)] * 2,
        compiler_params=pltpu.CompilerParams(
            dimension_semantics=("arbitrary", "arbitrary"),
        ),
    )(e3, n3)


def _tc_fixup_body(idxs_ref, x1c_ref, x2c_ref, ns_ref, es_ref,
                   x1_ref, x2_ref):
    del idxs_ref, x1c_ref, x2c_ref
    x1_ref[...] = ns_ref[...]
    x2_ref[...] = es_ref[...]


def _tc_fixup(idxs, x1c, x2c, e4, n4):
    sblk = (B, 1, 1, HW)
    grid_spec = pltpu.PrefetchScalarGridSpec(
        num_scalar_prefetch=1,
        grid=(K,),
        in_specs=[
            pl.BlockSpec(sblk, lambda k, r: (0, r[k], 0, 0)),
            pl.BlockSpec(sblk, lambda k, r: (0, r[L + k], 0, 0)),
            pl.BlockSpec(sblk, lambda k, r: (0, r[L + k], 0, 0)),
            pl.BlockSpec(sblk, lambda k, r: (0, r[k], 0, 0)),
        ],
        out_specs=[
            pl.BlockSpec(sblk, lambda k, r: (0, r[k], 0, 0)),
            pl.BlockSpec(sblk, lambda k, r: (0, r[L + k], 0, 0)),
        ],
    )
    return pl.pallas_call(
        _tc_fixup_body,
        grid_spec=grid_spec,
        out_shape=[jax.ShapeDtypeStruct(x1c.shape, x1c.dtype)] * 2,
        input_output_aliases={1: 0, 2: 1},
        compiler_params=pltpu.CompilerParams(
            dimension_semantics=("arbitrary",),
        ),
    )(idxs, x1c, x2c, n4, e4)


def kernel(e, n, bn_e, bn_n):
    idxs = _sc_topk(bn_e, bn_n)
    e3 = e.reshape(B, C, HW)
    n3 = n.reshape(B, C, HW)
    x1c, x2c = _tc_bulk(e3, n3)
    x1, x2 = _tc_fixup(
        idxs,
        x1c.reshape(B, C, 1, HW), x2c.reshape(B, C, 1, HW),
        e3.reshape(B, C, 1, HW), n3.reshape(B, C, 1, HW),
    )
    return (x1.reshape(e.shape), x2.reshape(e.shape))


# speedup vs baseline: 2.7427x; 2.7427x over previous
import functools

import jax
import jax.numpy as jnp
import numpy as np
from jax import lax
from jax.experimental import pallas as pl
from jax.experimental.pallas import tpu as pltpu
from jax.experimental.pallas import tpu_sc as plsc

C = 384
K = 5
L = 16
NSLICE = C // L
BIG = np.int32(1 << 30)
B = 16
HW = 56 * 56
CB = 96


def _sc_topk_body(bn_e_hbm, bn_n_hbm, idxs_hbm, vb_ref, tmp_ref, iout_ref):
    cid = lax.axis_index("c")
    sid = lax.axis_index("s")
    lane = lax.iota(jnp.int32, L)

    def splat_min_i32(x):
        for d in (1, 2, 4, 8):
            tmp_ref[...] = x
            x = jnp.minimum(x, plsc.load_gather(tmp_ref, [lane ^ d]))
        return x

    def splat_min_f32(x):
        for d in (1, 2, 4, 8):
            tmp_ref[...] = plsc.bitcast(x, jnp.int32)
            g = plsc.bitcast(plsc.load_gather(tmp_ref, [lane ^ d]),
                             jnp.float32)
            x = jnp.minimum(x, g)
        return x

    def top5_smallest():
        def absify(i, _):
            vb_ref[pl.ds(i * L, L)] = jnp.abs(vb_ref[pl.ds(i * L, L)])
            return 0
        lax.fori_loop(0, NSLICE, absify, 0)

        idxvec = jnp.zeros((L,), jnp.int32)
        for p in range(K):
            def scan_min(i, m):
                return jnp.minimum(m, vb_ref[pl.ds(i * L, L)])
            m = lax.fori_loop(0, NSLICE, scan_min,
                              jnp.full((L,), jnp.inf, jnp.float32))
            mmin = splat_min_f32(m)

            def scan_arg(i, best):
                v = vb_ref[pl.ds(i * L, L)]
                cand = jnp.where(v == mmin, lane + i * L, BIG)
                return jnp.minimum(best, cand)
            best = lax.fori_loop(0, NSLICE, scan_arg,
                                 jnp.full((L,), BIG, jnp.int32))
            widx = splat_min_i32(best)

            def mask_out(i, _):
                v = vb_ref[pl.ds(i * L, L)]
                vb_ref[pl.ds(i * L, L)] = jnp.where(
                    lane + i * L == widx, jnp.inf, v)
                return 0
            lax.fori_loop(0, NSLICE, mask_out, 0)
            idxvec = jnp.where(lane == p, widx, idxvec)
        return idxvec

    @pl.when(jnp.logical_and(cid == 0, sid == 0))
    def _():
        pltpu.sync_copy(bn_n_hbm, vb_ref)
        iout_ref[...] = top5_smallest()
        pltpu.sync_copy(iout_ref, idxs_hbm.at[pl.ds(L, L)])

    @pl.when(jnp.logical_and(cid != 0, sid == 0))
    def _():
        pltpu.sync_copy(bn_e_hbm, vb_ref)
        iout_ref[...] = top5_smallest()
        pltpu.sync_copy(iout_ref, idxs_hbm.at[pl.ds(0, L)])


def _sc_topk(bn_e, bn_n):
    mesh = plsc.VectorSubcoreMesh(core_axis_name="c", subcore_axis_name="s")
    fn = pl.kernel(
        _sc_topk_body,
        out_type=jax.ShapeDtypeStruct((2 * L,), jnp.int32),
        mesh=mesh,
        scratch_types=[
            pltpu.VMEM((C,), jnp.float32),
            pltpu.VMEM((L,), jnp.int32),
            pltpu.VMEM((L,), jnp.int32),
        ],
        compiler_params=pltpu.CompilerParams(needs_layout_passes=False),
    )
    return fn(bn_e, bn_n)


def _tc_exchange_body(idxs_ref, e_ref, n_ref, *rest):
    swap_refs = rest[:2 * K]
    x1_ref, x2_ref = rest[2 * K], rest[2 * K + 1]
    j = pl.program_id(1)
    base = j * CB
    x1_ref[...] = e_ref[...]
    x2_ref[...] = n_ref[...]
    for k in range(K):
        p1 = idxs_ref[k]
        p2 = idxs_ref[L + k]
        ns_ref = swap_refs[k]
        es_ref = swap_refs[K + k]

        @pl.when(jnp.logical_and(p1 >= base, p1 < base + CB))
        def _():
            x1_ref[:, pl.ds(p1 - base, 1)] = ns_ref[...].reshape(1, 1, HW)

        @pl.when(jnp.logical_and(p2 >= base, p2 < base + CB))
        def _():
            x2_ref[:, pl.ds(p2 - base, 1)] = es_ref[...].reshape(1, 1, HW)


def _tc_exchange(idxs, e3, n3):
    blk = (1, CB, HW)
    sblk = (1, 1, 1, HW)
    nswap_specs = [
        pl.BlockSpec(sblk, (lambda b, j, r, k=k: (b, r[L + k], 0, 0)))
        for k in range(K)
    ]
    eswap_specs = [
        pl.BlockSpec(sblk, (lambda b, j, r, k=k: (b, r[k], 0, 0)))
        for k in range(K)
    ]
    grid_spec = pltpu.PrefetchScalarGridSpec(
        num_scalar_prefetch=1,
        grid=(B, C // CB),
        in_specs=[
            pl.BlockSpec(blk, lambda b, j, r: (b, j, 0)),
            pl.BlockSpec(blk, lambda b, j, r: (b, j, 0)),
            *nswap_specs,
            *eswap_specs,
        ],
        out_specs=[
            pl.BlockSpec(blk, lambda b, j, r: (b, j, 0)),
            pl.BlockSpec(blk, lambda b, j, r: (b, j, 0)),
        ],
    )
    call = pl.pallas_call(
        _tc_exchange_body,
        grid_spec=grid_spec,
        out_shape=[jax.ShapeDtypeStruct(e3.shape, e3.dtype)] * 2,
        compiler_params=pltpu.CompilerParams(
            dimension_semantics=("arbitrary", "arbitrary"),
        ),
    )
    n4 = n3.reshape(B, C, 1, HW)
    e4 = e3.reshape(B, C, 1, HW)
    return call(idxs, e3, n3, *([n4] * K), *([e4] * K))


def _tc_bulk_body(e_ref, n_ref, x1_ref, x2_ref):
    x1_ref[...] = e_ref[...]
    x2_ref[...] = n_ref[...]


def _tc_bulk(e3, n3):
    blk = (1, CB, HW)
    grid_spec = pl.GridSpec(
        grid=(B, C // CB),
        in_specs=[
            pl.BlockSpec(blk, lambda b, j: (b, j, 0)),
            pl.BlockSpec(blk, lambda b, j: (b, j, 0)),
        ],
        out_specs=[
            pl.BlockSpec(blk, lambda b, j: (b, j, 0)),
            pl.BlockSpec(blk, lambda b, j: (b, j, 0)),
        ],
    )
    return pl.pallas_call(
        _tc_bulk_body,
        grid_spec=grid_spec,
        out_shape=[jax.ShapeDtypeStruct(e3.shape, e3.dtype)] * 2,
        compiler_params=pltpu.CompilerParams(
            dimension_semantics=("arbitrary", "arbitrary"),
        ),
    )(e3, n3)


def _tc_paybuild_body(idxs_ref, nwin_ref, ewin_ref, pay_ref):
    i = pl.program_id(0)

    @pl.when(i < K)
    def _():
        col = idxs_ref[L + i] % 8
        pay_ref[...] = nwin_ref[:, pl.ds(col, 1)].reshape(B, 1, 1, HW)

    @pl.when(i >= K)
    def _():
        col = idxs_ref[i - K] % 8
        pay_ref[...] = ewin_ref[:, pl.ds(col, 1)].reshape(B, 1, 1, HW)


def _tc_paybuild(idxs, e3, n3):
    wblk = (B, 8, HW)
    grid_spec = pltpu.PrefetchScalarGridSpec(
        num_scalar_prefetch=1,
        grid=(2 * K,),
        in_specs=[
            pl.BlockSpec(
                wblk,
                lambda i, r: (0, r[L + jnp.minimum(i, K - 1)] // 8, 0)),
            pl.BlockSpec(
                wblk,
                lambda i, r: (0, r[jnp.maximum(i, K) - K] // 8, 0)),
        ],
        out_specs=[
            pl.BlockSpec((B, 1, 1, HW), lambda i, r: (0, i, 0, 0)),
        ],
    )
    return pl.pallas_call(
        _tc_paybuild_body,
        grid_spec=grid_spec,
        out_shape=[jax.ShapeDtypeStruct((B, 2 * K, 1, HW), jnp.float32)],
        compiler_params=pltpu.CompilerParams(
            dimension_semantics=("arbitrary",),
        ),
    )(idxs, n3, e3)[0]


def _tc_main_body(idxs_ref, e_ref, n_ref, pay_ref, x1_ref, x2_ref):
    j = pl.program_id(1)
    base = j * CB
    x1_ref[...] = e_ref[...]
    x2_ref[...] = n_ref[...]
    for k in range(K):
        p1 = idxs_ref[k]
        p2 = idxs_ref[L + k]

        @pl.when(jnp.logical_and(p1 >= base, p1 < base + CB))
        def _():
            x1_ref[:, pl.ds(p1 - base, 1)] = (
                pay_ref[:, k:k + 1].reshape(1, 1, HW))

        @pl.when(jnp.logical_and(p2 >= base, p2 < base + CB))
        def _():
            x2_ref[:, pl.ds(p2 - base, 1)] = (
                pay_ref[:, K + k:K + k + 1].reshape(1, 1, HW))


def _tc_main(idxs, e3, n3, pay):
    blk = (1, CB, HW)
    grid_spec = pltpu.PrefetchScalarGridSpec(
        num_scalar_prefetch=1,
        grid=(B, C // CB),
        in_specs=[
            pl.BlockSpec(blk, lambda b, j, r: (b, j, 0)),
            pl.BlockSpec(blk, lambda b, j, r: (b, j, 0)),
            pl.BlockSpec((1, 2 * K, 1, HW), lambda b, j, r: (b, 0, 0, 0)),
        ],
        out_specs=[
            pl.BlockSpec(blk, lambda b, j, r: (b, j, 0)),
            pl.BlockSpec(blk, lambda b, j, r: (b, j, 0)),
        ],
    )
    return pl.pallas_call(
        _tc_main_body,
        grid_spec=grid_spec,
        out_shape=[jax.ShapeDtypeStruct(e3.shape, e3.dtype)] * 2,
        compiler_params=pltpu.CompilerParams(
            dimension_semantics=("arbitrary", "arbitrary"),
        ),
    )(idxs, e3, n3, pay)


def kernel(e, n, bn_e, bn_n):
    idxs = _sc_topk(bn_e, bn_n)
    e3 = e.reshape(B, C, HW)
    n3 = n.reshape(B, C, HW)
    pay = _tc_paybuild(idxs, e3, n3)
    x1, x2 = _tc_main(idxs, e3, n3, pay)
    return (x1.reshape(e.shape), x2.reshape(e.shape))
